# baseline (speedup 1.0000x reference)
import functools

import jax
import jax.numpy as jnp
from jax import lax
from jax.experimental import pallas as pl
from jax.experimental.pallas import tpu as pltpu
from jax.experimental.pallas import tpu_sc as plsc

N = 4096
E = 65536
M = N * (N - 1) // 2

NC, NS, L = 2, 16, 16
NW = NC * NS
EPW = E // NW
VPW = EPW // L
CHW = 128
NCHUNK = EPW // CHW

S_A = 0x7FC00001
S_AB = 0x7FC00003

_mesh = plsc.VectorSubcoreMesh(
    core_axis_name="c", subcore_axis_name="s", num_cores=NC, num_subcores=NS
)
_params = pltpu.CompilerParams(needs_layout_passes=False)

_NF = 8
_FBW = M // _NF
_tc_mesh = pltpu.create_tensorcore_mesh("tc")


@functools.partial(
    pl.kernel,
    out_type=(),
    mesh=_tc_mesh,
    scratch_types=[
        pltpu.VMEM((_FBW,), jnp.float32),
        pltpu.SMEM((1,), jnp.float32),
        pltpu.SemaphoreType.DMA,
    ],
)
def _fill(v_hbm, out_hbm, buf, v_smem, sem):
    pltpu.sync_copy(v_hbm, v_smem)
    buf[...] = jnp.full((_FBW,), v_smem[0], jnp.float32)
    for i in range(_NF):
        pltpu.make_async_copy(
            buf, out_hbm.at[pl.ds(i * _FBW, _FBW)], sem).start()
    for i in range(_NF):
        pltpu.make_async_copy(
            buf, out_hbm.at[pl.ds(i * _FBW, _FBW)], sem).wait()


def _stage_edges(edge_hbm, base, src_v, dst_v):
    pltpu.sync_copy(edge_hbm.at[0, pl.ds(base, EPW)], src_v)
    pltpu.sync_copy(edge_hbm.at[1, pl.ds(base, EPW)], dst_v)


def _compact_idx(src_v, dst_v, mflat, m2d):
    def p1(i, off):
        sv = src_v[pl.ds(i * L, L)]
        dv = dst_v[pl.ds(i * L, L)]
        valid = sv < dv
        a = sv * (2 * N - 1 - sv)
        m = (a >> 1) + dv - sv - 1
        plsc.store_compressed(mflat.at[pl.ds(off, L)], m, mask=valid)
        return off + jnp.sum(valid.astype(jnp.int32))
    nvalid = lax.fori_loop(0, VPW, p1, jnp.int32(0))
    nch = (nvalid + CHW - 1) // CHW

    @pl.when(nvalid > 0)
    def _():
        def fill(wi, c):
            start = wi * L
            pos = start + lax.iota(jnp.int32, L)
            sel = pos % nvalid
            mflat[pl.ds(start, L)] = plsc.load_gather(mflat, [sel])
            return c
        lax.fori_loop(nvalid // L, (nch * CHW) // L, fill, jnp.int32(0))

        def crow(k, c):
            j = k // (CHW // L)
            col = (k % (CHW // L)) * L
            m2d[j, pl.ds(col, L)] = mflat[pl.ds(k * L, L)]
            return c
        lax.fori_loop(0, nch * (CHW // L), crow, jnp.int32(0))
    return nch


def _fire(nch, mk):
    def go(j, c):
        mk(j).start()
        return c
    lax.fori_loop(0, nch, go, jnp.int32(0))


def _drain(nch, mk):
    def go(j, c):
        mk(j).wait()
        return c
    lax.fori_loop(0, nch, go, jnp.int32(0))


def _gather_mk(out_hbm, mflat, gflat, sem):
    return lambda j: pltpu.make_async_copy(
        out_hbm.at[mflat.at[pl.ds(j * CHW, CHW)]],
        gflat.at[pl.ds(j * CHW, CHW)],
        sem,
    )


def _scatter_mk(out_hbm, m2d, vflat, sem):
    return lambda j: pltpu.make_async_copy(
        vflat.at[pl.ds(j * CHW, CHW)],
        out_hbm.at[m2d.at[j]],
        sem,
    )


def _cross_core_barrier(flags_hbm, fl_v, phase, cid):
    plsc.subcore_barrier()
    sid = lax.axis_index("s")

    @pl.when(sid == 0)
    def _():
        fl_v[pl.ds(0, L)] = jnp.full((L,), phase, jnp.int32)
        pltpu.sync_copy(fl_v, flags_hbm.at[cid])

        def not_done(c):
            return c < L

        def poll(c):
            pltpu.sync_copy(flags_hbm.at[1 - cid], fl_v)
            o = fl_v[pl.ds(0, L)]
            return jnp.sum((o >= phase).astype(jnp.int32))
        lax.while_loop(not_done, poll, jnp.int32(0))
    plsc.subcore_barrier()


@functools.partial(
    pl.kernel,
    out_type=(),
    mesh=_mesh,
    compiler_params=_params,
    scratch_types=[
        pltpu.VMEM((EPW,), jnp.int32),
        pltpu.VMEM((EPW,), jnp.int32),
        pltpu.VMEM((EPW,), jnp.int32),
        pltpu.VMEM((NCHUNK, CHW), jnp.int32),
        pltpu.VMEM((EPW,), jnp.int32),
        pltpu.VMEM((NCHUNK, CHW), jnp.int32),
        pltpu.VMEM((EPW,), jnp.float32),
        pltpu.VMEM((EPW,), jnp.float32),
        pltpu.VMEM((CHW,), jnp.float32),
        pltpu.VMEM((4, L), jnp.float32),
        pltpu.VMEM((L,), jnp.int32),
        pltpu.SemaphoreType.DMA,
    ],
)
def _scmain(edgea_hbm, edgeb_hbm, vals_hbm, out_hbm, flags_hbm,
            src_v, dst_v, mflat_a, m2d_a, mflat_b, m2d_b,
            gflat, vflat, val_row, vals_v, fl_v, sem):
    cid = lax.axis_index("c")
    wid = lax.axis_index("s") * NC + cid
    base = pl.multiple_of(wid * EPW, EPW)

    _stage_edges(edgea_hbm, base, src_v, dst_v)
    nch_a = _compact_idx(src_v, dst_v, mflat_a, m2d_a)
    sa = plsc.bitcast(jnp.full((L,), S_A, jnp.int32), jnp.float32)
    for k in range(CHW // L):
        val_row[pl.ds(k * L, L)] = sa
    _fire(nch_a, lambda j: pltpu.make_async_copy(
        val_row, out_hbm.at[m2d_a.at[j]], sem))

    pltpu.sync_copy(vals_hbm, vals_v)
    _stage_edges(edgeb_hbm, base, src_v, dst_v)
    nch_b = _compact_idx(src_v, dst_v, mflat_b, m2d_b)
    _drain(nch_a, lambda j: pltpu.make_async_copy(
        val_row, out_hbm.at[m2d_a.at[j]], sem))

    _cross_core_barrier(flags_hbm, fl_v, 1, cid)

    v10 = vals_v[1, :]
    gmk_b = _gather_mk(out_hbm, mflat_b, gflat, sem)
    _fire(nch_b, gmk_b)
    _drain(nch_b, gmk_b)

    def conv_b(k, c, v10=v10):
        gi = plsc.bitcast(gflat[pl.ds(k * L, L)], jnp.int32)
        hit = (gi == S_A) | (gi == S_AB)
        sab = plsc.bitcast(jnp.full((L,), S_AB, jnp.int32), jnp.float32)
        vflat[pl.ds(k * L, L)] = jnp.where(hit, sab, v10)
        return c
    lax.fori_loop(0, nch_b * (CHW // L), conv_b, jnp.int32(0))
    smk_b = _scatter_mk(out_hbm, m2d_b, vflat, sem)
    _fire(nch_b, smk_b)
    _drain(nch_b, smk_b)

    _cross_core_barrier(flags_hbm, fl_v, 2, cid)

    v01 = vals_v[0, :]
    v11 = vals_v[2, :]
    gmk_a = _gather_mk(out_hbm, mflat_a, gflat, sem)
    _fire(nch_a, gmk_a)
    _drain(nch_a, gmk_a)

    def conv_a(k, c, v01=v01, v11=v11):
        g = gflat[pl.ds(k * L, L)]
        gi = plsc.bitcast(g, jnp.int32)
        nv = jnp.where(gi == S_A, v01, jnp.where(gi == S_AB, v11, g))
        vflat[pl.ds(k * L, L)] = nv
        return c
    lax.fori_loop(0, nch_a * (CHW // L), conv_a, jnp.int32(0))
    smk_a = _scatter_mk(out_hbm, m2d_a, vflat, sem)
    _fire(nch_a, smk_a)
    _drain(nch_a, smk_a)


def kernel(edge_index, t, Qt, ref_edge_index):
    t0 = t[0].astype(jnp.int32)
    Q0 = Qt[0]
    Qp = lax.dynamic_index_in_dim(Qt, t0 - 1, 0, keepdims=False)
    Qe = lax.dynamic_index_in_dim(Qt, t0, 0, keepdims=False)
    v00 = (Q0[1, 0] * Qp[0, 1] / Qe[0, 0]).reshape(1)
    v01 = Q0[1, 1] * Qp[0, 1] / Qe[0, 1]
    v10 = Q0[1, 0] * Qp[1, 1] / Qe[1, 0]
    v11 = Q0[1, 1] * Qp[1, 1] / Qe[1, 1]
    vals = jnp.broadcast_to(
        jnp.stack([v01, v10, v11, v11])[:, None], (4, L))

    out_ref = pl.empty_ref_like(pltpu.HBM((M,), jnp.float32))
    flags_ref = jax.new_ref(jnp.full((2, L), -1, jnp.int32))
    _fill(v00, out_ref)
    _scmain(edge_index, ref_edge_index, vals, out_ref, flags_ref)
    return jax.freeze(out_ref)
